# Initial kernel scaffold; baseline (speedup 1.0000x reference)
import jax
import jax.numpy as jnp
from jax.experimental import pallas as pl


def kernel(des1, det1, qlt1, des2, det2, qlt2, aflow):
    raise NotImplementedError("write your pallas kernel here")


# speedup vs baseline: 1.7262x; 1.7262x over previous
import functools

import jax
import jax.numpy as jnp
import numpy as np
from jax import lax
from jax.experimental import pallas as pl
from jax.experimental.pallas import tpu as pltpu
from jax.experimental.pallas import tpu_sc as plsc

B, D, H, W = 4, 128, 224, 224
HW = H * W
M = 16
CD = 8
NCELL = 24
NSAMP = B * NCELL * NCELL
POS_R = 3

NC, NS = 2, 16
NWORK = NC * NS
SPW = NSAMP // NWORK
STARTS = (0, 16, 32, 48, 56)

_pos = np.array([(i, j) for i in range(-POS_R, POS_R + 1)
                 for j in range(-POS_R, POS_R + 1)
                 if i * i + j * j <= POS_R * POS_R], dtype=np.int64).reshape(-1, 2).T
_neg = np.array([(i, j) for i in range(-8, 9, 2) for j in range(-8, 9, 2)
                 if 49 <= i * i + j * j <= 64], dtype=np.int64).reshape(-1, 2).T
NPOS = _pos.shape[1]
NNEG = _neg.shape[1]
NOFF = NPOS + NNEG
_offs = np.concatenate([_pos, _neg], axis=1).astype(np.int32)

ROWS_PW = SPW * NOFF
PN_NCH = 9
PN_ROWS = ROWS_PW // PN_NCH

_bv = np.repeat(np.arange(B), NCELL * NCELL).astype(np.int32)
_cy = np.tile(np.repeat(np.arange(NCELL), NCELL), B).astype(np.int32)
_cx = np.tile(np.arange(NCELL), B * NCELL).astype(np.int32)
C_ROWS = np.zeros((4, NSAMP), np.int32)
C_ROWS[0] = _bv * HW
C_ROWS[1] = M + 8 * _cx
C_ROWS[2] = M + 8 * _cy
C_ROWS[3] = _bv


def _tc_sample_body(u1, g1, u2, g2, c, arows):
    iota = lax.broadcasted_iota(jnp.int32, (NSAMP, 64), 1)
    bb = c[0, :]
    colb = c[1, :]
    rowb = c[2, :]

    def samp(u, g):
        v = jnp.log(u[...] + 1e-12) + g[...]
        vmax = jnp.max(v, axis=1, keepdims=True)
        idx = jnp.min(jnp.where(v == vmax, iota, 64), axis=1)
        col = colb + (idx % 8)
        row = rowb + (idx // 8)
        return col, row

    y1, x1 = samp(u1, g1)
    yd, xd = samp(u2, g2)
    arows[pl.ds(0 * NSAMP, NSAMP)] = bb + y1 * W + x1
    arows[pl.ds(1 * NSAMP, NSAMP)] = bb + yd * W + xd
    arows[pl.ds(2 * NSAMP, NSAMP)] = xd
    arows[pl.ds(3 * NSAMP, NSAMP)] = yd
    arows[pl.ds(4 * NSAMP, NSAMP)] = c[3, :]


def _tc_sample(u1, g1, u2, g2, c_rows):
    return pl.pallas_call(
        _tc_sample_body,
        out_shape=jax.ShapeDtypeStruct((5 * NSAMP,), jnp.int32),
    )(u1, g1, u2, g2, c_rows)


@functools.cache
def _sc_kernels():
    mesh = plsc.VectorSubcoreMesh(core_axis_name="c", subcore_axis_name="s",
                                  num_cores=NC, num_subcores=NS)

    @functools.partial(
        pl.kernel,
        out_type=(jax.ShapeDtypeStruct((NSAMP, D), jnp.float32),
                  jax.ShapeDtypeStruct((NSAMP, D), jnp.float32),
                  jax.ShapeDtypeStruct((NSAMP * NOFF, D), jnp.float32),
                  jax.ShapeDtypeStruct((3 * NSAMP,), jnp.int32),
                  jax.ShapeDtypeStruct((NSAMP,), jnp.float32)),
        mesh=mesh,
        scratch_types=[
            pltpu.VMEM((SPW,), jnp.int32),
            pltpu.VMEM((SPW,), jnp.int32),
            pltpu.VMEM((SPW, D), jnp.float32),
            pltpu.VMEM((SPW, D), jnp.float32),
            pltpu.VMEM((SPW,), jnp.float32),
            pltpu.VMEM((SPW,), jnp.float32),
            pltpu.VMEM((SPW,), jnp.float32),
            pltpu.VMEM((ROWS_PW,), jnp.int32),
            pltpu.VMEM((SPW,), jnp.int32),
            pltpu.VMEM((SPW,), jnp.int32),
            pltpu.VMEM((SPW,), jnp.int32),
            pltpu.VMEM((PN_ROWS, D), jnp.float32),
            pltpu.VMEM((PN_ROWS, D), jnp.float32),
            pltpu.SemaphoreType.DMA,
            pltpu.SemaphoreType.DMA,
        ],
    )
    def sc_gather(des1_t, des2_t, af0f, af1f, q1f, acols,
                  s_out, d_out, pn_out, meta_out, q1_out,
                  f1v, fdv, s1v, drv, a0v, a1v, qv, idxv, x0v, x1v, inbv,
                  pnv0, pnv1, sem0, sem1):
        wid = lax.axis_index("s") * NC + lax.axis_index("c")
        base = wid * SPW
        bbase = (base // (NCELL * NCELL)) * HW

        pltpu.sync_copy(acols.at[pl.ds(0 * NSAMP + base, SPW)], f1v)
        pltpu.sync_copy(acols.at[pl.ds(1 * NSAMP + base, SPW)], fdv)

        cp1 = pltpu.async_copy(des1_t.at[f1v], s1v, sem0)
        cp2 = pltpu.async_copy(des2_t.at[fdv], drv, sem1)
        cp1.wait()
        pltpu.sync_copy(s1v, s_out.at[pl.ds(base, SPW), :])
        cp2.wait()
        pltpu.sync_copy(drv, d_out.at[pl.ds(base, SPW), :])

        cp3 = pltpu.async_copy(af0f.at[f1v], a0v, sem0)
        cp4 = pltpu.async_copy(af1f.at[f1v], a1v, sem1)
        cp3.wait()
        cp4.wait()
        cp5 = pltpu.async_copy(q1f.at[f1v], qv, sem0)

        for st in STARTS:
            a0 = a0v[pl.ds(st, 16)]
            a1 = a1v[pl.ds(st, 16)]
            xy0 = (a0 + 0.5).astype(jnp.int32)
            xy1 = (a1 + 0.5).astype(jnp.int32)
            inb = jnp.where((xy0 >= 0) & (xy1 >= 0) & (xy0 < W) & (xy1 < H),
                            1, 0)
            x0v[pl.ds(st, 16)] = xy0
            x1v[pl.ds(st, 16)] = xy1
            inbv[pl.ds(st, 16)] = inb
            for o in range(NOFF):
                gx = jnp.clip(xy0 + int(_offs[0, o]), 0, W - 1)
                gy = jnp.clip(xy1 + int(_offs[1, o]), 0, H - 1)
                idxv[pl.ds(o * SPW + st, 16)] = bbase + gy * W + gx

        pltpu.sync_copy(x0v, meta_out.at[pl.ds(0 * NSAMP + base, SPW)])
        pltpu.sync_copy(x1v, meta_out.at[pl.ds(1 * NSAMP + base, SPW)])
        pltpu.sync_copy(inbv, meta_out.at[pl.ds(2 * NSAMP + base, SPW)])
        cp5.wait()
        pltpu.sync_copy(qv, q1_out.at[pl.ds(base, SPW)])

        bufs = (pnv0, pnv1)
        sems = (sem0, sem1)
        cps = [None, None]
        for t in range(PN_NCH):
            k = t % 2
            cps[k] = pltpu.async_copy(
                des2_t.at[idxv.at[pl.ds(t * PN_ROWS, PN_ROWS)]], bufs[k],
                sems[k])
            if t > 0:
                cps[1 - k].wait()
                pltpu.sync_copy(
                    bufs[1 - k],
                    pn_out.at[pl.ds(base * NOFF + (t - 1) * PN_ROWS, PN_ROWS), :])
        last = (PN_NCH - 1) % 2
        cps[last].wait()
        pltpu.sync_copy(
            bufs[last],
            pn_out.at[pl.ds(base * NOFF + (PN_NCH - 1) * PN_ROWS, PN_ROWS), :])

    @functools.partial(
        pl.kernel,
        out_type=jax.ShapeDtypeStruct((NSAMP,), jnp.float32),
        mesh=mesh,
        scratch_types=[
            pltpu.VMEM((SPW,), jnp.int32),
            pltpu.VMEM((SPW,), jnp.float32),
            pltpu.VMEM((SPW,), jnp.float32),
            pltpu.VMEM((SPW,), jnp.float32),
            pltpu.SemaphoreType.DMA,
        ],
    )
    def sc_qlt(q2f, selflat, q1g, q_out, sv, q2v, q1v, rv, sem):
        wid = lax.axis_index("s") * NC + lax.axis_index("c")
        base = wid * SPW
        pltpu.sync_copy(selflat.at[pl.ds(base, SPW)], sv)
        pltpu.sync_copy(q1g.at[pl.ds(base, SPW)], q1v)
        pltpu.async_copy(q2f.at[sv], q2v, sem).wait()
        for st in STARTS:
            q1 = q1v[pl.ds(st, 16)]
            q2 = q2v[pl.ds(st, 16)]
            rv[pl.ds(st, 16)] = (q1 + q2) * 0.5
        pltpu.sync_copy(rv, q_out.at[pl.ds(base, SPW)])

    return sc_gather, sc_qlt


_RB = 288
_GRP = 4
_NSTEP = NSAMP // _RB
_NCOL = 1 + NNEG + NSAMP


def _tc_main_body(s_ref, pn_ref, dr_ref, meta3_ref, ar3_ref, ar2_ref,
                  scores_ref, small_ref, self_ref):
    s = pl.program_id(0)
    s1 = s_ref[...]
    pn4 = pn_ref[...].reshape(_GRP, NOFF, SPW, D)
    s1g = s1.reshape(_GRP, SPW, D)
    ps = jnp.sum(pn4 * s1g[:, None, :, :], axis=3)
    psp = ps[:, :NPOS, :]
    nsc = ps[:, NPOS:NOFF, :]
    pmax = jnp.max(psp, axis=1)
    io = lax.broadcasted_iota(jnp.int32, (_GRP, NPOS, SPW), 1)
    post = jnp.min(jnp.where(psp == pmax[:, None, :], io, NPOS + 70), axis=1)
    onehot = io == post[:, None, :]
    seli = jnp.zeros((_GRP, SPW), jnp.int32)
    selj = jnp.zeros((_GRP, SPW), jnp.int32)
    for o in range(NPOS):
        seli = seli + jnp.where(onehot[:, o, :], int(_offs[0, o]), 0)
        selj = selj + jnp.where(onehot[:, o, :], int(_offs[1, o]), 0)

    xy0g = meta3_ref[0, pl.ds(s * _GRP, _GRP), :]
    xy1g = meta3_ref[1, pl.ds(s * _GRP, _GRP), :]
    browg = ar3_ref[4, pl.ds(s * _GRP, _GRP), :]
    selx = jnp.clip(xy0g + seli, 0, W - 1)
    sely = jnp.clip(xy1g + selj, 0, H - 1)
    self_ref[0, :, :] = browg * HW + sely * W + selx

    dsc = lax.dot_general(s1, dr_ref[...], (((1,), (1,)), ((), ())),
                          preferred_element_type=jnp.float32)
    dscg = dsc.reshape(_GRP, SPW, NSAMP)
    xdr = ar2_ref[2:3, :][:, None, :]
    ydr = ar2_ref[3:4, :][:, None, :]
    bdr = ar2_ref[4:5, :][:, None, :]
    dx = xdr - xy0g[:, :, None]
    dy = ydr - xy1g[:, :, None]
    dis2 = dx * dx + dy * dy + jnp.where(bdr != browg[:, :, None],
                                         POS_R * POS_R, 0)
    dscm = jnp.where(dis2 < POS_R * POS_R, 0.0, dscg)

    scores_ref[...] = dscm
    small_ref[0, 0, :, :] = pmax
    for o in range(NNEG):
        small_ref[0, 1 + o, :, :] = nsc[:, o, :]


def _tc_main(s_des1, pn, distr, meta3, ar3, ar2):
    return pl.pallas_call(
        _tc_main_body,
        grid=(_NSTEP,),
        in_specs=[
            pl.BlockSpec((_RB, D), lambda s: (s, 0)),
            pl.BlockSpec((_RB * NOFF, D), lambda s: (s, 0)),
            pl.BlockSpec((NSAMP, D), lambda s: (0, 0)),
            pl.BlockSpec((3, NWORK, SPW), lambda s: (0, 0, 0)),
            pl.BlockSpec((5, NWORK, SPW), lambda s: (0, 0, 0)),
            pl.BlockSpec((5, NSAMP), lambda s: (0, 0)),
        ],
        out_specs=[
            pl.BlockSpec((_GRP, SPW, NSAMP), lambda s: (s, 0, 0)),
            pl.BlockSpec((1, 1 + NNEG, _GRP, SPW), lambda s: (s, 0, 0, 0)),
            pl.BlockSpec((1, _GRP, SPW), lambda s: (s, 0, 0)),
        ],
        out_shape=(jax.ShapeDtypeStruct((NWORK, SPW, NSAMP), jnp.float32),
                   jax.ShapeDtypeStruct((_NSTEP, 1 + NNEG, _GRP, SPW),
                                        jnp.float32),
                   jax.ShapeDtypeStruct((_NSTEP, _GRP, SPW), jnp.int32)),
    )(s_des1, pn, distr, meta3, ar3, ar2)


def kernel(des1, det1, qlt1, des2, det2, qlt2, aflow):
    f32 = jnp.float32

    def unshuffle(det):
        return det[:, 0, M:H - M, M:W - M].reshape(B, NCELL, CD, NCELL, CD) \
                  .transpose(0, 1, 3, 2, 4).reshape(NSAMP, 64)

    k1, k2 = jax.random.split(jax.random.key(42))
    u1 = unshuffle(det1)
    u2 = unshuffle(det2)
    g1 = jax.random.gumbel(k1, (B, NCELL, NCELL, 64), f32).reshape(NSAMP, 64)
    g2 = jax.random.gumbel(k2, (B, NCELL, NCELL, 64), f32).reshape(NSAMP, 64)

    arows = _tc_sample(u1, g1, u2, g2, jnp.asarray(C_ROWS))

    des1_t = des1.transpose(0, 2, 3, 1).reshape(B * HW, D)
    des2_t = des2.transpose(0, 2, 3, 1).reshape(B * HW, D)
    af0f = aflow[:, 0].reshape(-1)
    af1f = aflow[:, 1].reshape(-1)
    q1f = qlt1[:, 0].reshape(-1)
    q2f = qlt2[:, 0].reshape(-1)

    sc_gather, sc_qlt = _sc_kernels()
    s_des1, distr, pn, meta, q1g = sc_gather(des1_t, des2_t, af0f, af1f,
                                             q1f, arows)

    dsc_out, small, selflat = _tc_main(s_des1, pn, distr,
                                       meta.reshape(3, NWORK, SPW),
                                       arows.reshape(5, NWORK, SPW),
                                       arows.reshape(5, NSAMP))

    qlt = sc_qlt(q2f, selflat.reshape(NSAMP), q1g)

    sm = jnp.transpose(small, (0, 2, 3, 1)).reshape(NSAMP, 1 + NNEG)
    scores = jnp.concatenate([sm, dsc_out.reshape(NSAMP, NSAMP)], axis=1)
    labels = jnp.zeros(scores.shape, bool).at[:, :1].set(True)
    mask = meta.reshape(3, NSAMP)[2].astype(bool).reshape(B, NCELL * NCELL)
    return scores, labels, mask, qlt[:, None]
